# baseline (speedup 1.0000x reference)
import jax
import jax.numpy as jnp
from jax import lax
from jax.experimental import pallas as pl
from jax.experimental.pallas import tpu as pltpu
from jax.experimental.pallas import tpu_sc as plsc

B = 16384
EMB = 64
NW = 32
ROWS_PER_W = B // NW
OH = 80
UNROLL = 32


def _sc_gather_body(idx_hbm, table_hbm, out_hbm, idx_v, rows_v, sem):
    wid = lax.axis_index("s") * 2 + lax.axis_index("c")
    base = wid * ROWS_PER_W
    pltpu.sync_copy(idx_hbm.at[pl.ds(base, ROWS_PER_W)], idx_v)

    def step(i, carry):
        s = i * UNROLL
        vec = idx_v[pl.ds(s, UNROLL)]
        cps = []
        for j in range(UNROLL):
            r = vec[j]
            cps.append(pltpu.async_copy(
                table_hbm.at[pl.ds(r, 1)], rows_v.at[pl.ds(s + j, 1)], sem))
        for cp in cps:
            cp.wait()
        return carry

    lax.fori_loop(0, ROWS_PER_W // UNROLL, step, 0)
    pltpu.sync_copy(rows_v, out_hbm.at[pl.ds(base, ROWS_PER_W)])


def _make_sc_gather():
    return pl.kernel(
        _sc_gather_body,
        out_type=jax.ShapeDtypeStruct((B, EMB), jnp.float32),
        mesh=plsc.VectorSubcoreMesh(core_axis_name="c", subcore_axis_name="s"),
        scratch_types=[
            pltpu.VMEM((ROWS_PER_W,), jnp.int32),
            pltpu.VMEM((ROWS_PER_W, EMB), jnp.float32),
            pltpu.SemaphoreType.DMA,
        ],
    )


def _tc_body(id_emb_ref, sec_ref, stg_ref, reg_ref, numT_ref, tbdT_ref,
             w1aT_ref, w1mT_ref, w1nT_ref, b1_ref, g1_ref, be1_ref,
             w2T_ref, b2_ref, g2_ref, be2_ref, outT_ref):
    f32 = jnp.float32
    id_emb = id_emb_ref[:]
    iota = lax.broadcasted_iota(jnp.int32, (OH, B), 0)
    ohT = (jnp.where(iota == sec_ref[:], 1.0, 0.0)
           + jnp.where(iota == stg_ref[:], 1.0, 0.0)
           + jnp.where(iota == reg_ref[:], 1.0, 0.0)).astype(f32)
    mT = jnp.dot(w1mT_ref[:], tbdT_ref[:], preferred_element_type=f32)
    p1 = (lax.dot_general(w1aT_ref[:], id_emb, (((1,), (1,)), ((), ())),
                          preferred_element_type=f32)
          + jnp.dot(mT, ohT, preferred_element_type=f32)
          + jnp.dot(w1nT_ref[:], numT_ref[:], preferred_element_type=f32)
          + b1_ref[:])
    h = jnp.maximum(p1, 0.0)
    mu = jnp.mean(h, axis=1, keepdims=True)
    var = jnp.mean((h - mu) * (h - mu), axis=1, keepdims=True)
    h = (h - mu) / jnp.sqrt(var + 1e-5) * g1_ref[:] + be1_ref[:]
    p2 = jnp.dot(w2T_ref[:], h, preferred_element_type=f32) + b2_ref[:]
    h2 = jnp.maximum(p2, 0.0)
    mu2 = jnp.mean(h2, axis=1, keepdims=True)
    var2 = jnp.mean((h2 - mu2) * (h2 - mu2), axis=1, keepdims=True)
    h2 = (h2 - mu2) / jnp.sqrt(var2 + 1e-5) * g2_ref[:] + be2_ref[:]
    nrm = jnp.sqrt(jnp.sum(h2 * h2, axis=0, keepdims=True))
    outT_ref[:] = h2 / jnp.maximum(nrm, 1e-12)


_tc_mlp = pl.pallas_call(
    _tc_body,
    out_shape=jax.ShapeDtypeStruct((EMB, B), jnp.float32),
)


def kernel(id, sector, stage, region, deal_size, revenue_multiple, growth_rate,
           profitability, team_experience, market_size, deal_table,
           sector_table, stage_table, region_table, W1, b1, g1, be1,
           W2, b2, g2, be2):
    id_emb = _make_sc_gather()(id.astype(jnp.int32), deal_table)

    numT = jnp.stack([deal_size, revenue_multiple, growth_rate, profitability,
                      team_experience, market_size], axis=0).astype(jnp.float32)
    numT = jnp.pad(numT, ((0, 2), (0, 0)))
    w1nT = jnp.pad(W1[112:118], ((0, 2), (0, 0))).T

    tbdT = jnp.zeros((48, OH), dtype=jnp.float32)
    tbdT = tbdT.at[0:16, 0:50].set(sector_table.T)
    tbdT = tbdT.at[16:32, 50:60].set(stage_table.T)
    tbdT = tbdT.at[32:48, 60:80].set(region_table.T)

    sec = sector.astype(jnp.int32).reshape(1, B)
    stg = stage.astype(jnp.int32).reshape(1, B) + 50
    reg = region.astype(jnp.int32).reshape(1, B) + 60

    outT = _tc_mlp(
        id_emb, sec, stg, reg, numT, tbdT,
        W1[0:64].T, W1[64:112].T, w1nT,
        b1.reshape(128, 1), g1.reshape(128, 1), be1.reshape(128, 1),
        W2.T, b2.reshape(64, 1), g2.reshape(64, 1), be2.reshape(64, 1),
    )
    return outT.T


# speedup vs baseline: 1.0105x; 1.0105x over previous
import jax
import jax.numpy as jnp
from jax import lax
from jax.experimental import pallas as pl
from jax.experimental.pallas import tpu as pltpu
from jax.experimental.pallas import tpu_sc as plsc

B = 16384
EMB = 64
NW = 32
ROWS_PER_W = B // NW
OH = 80
UNROLL = 64


def _sc_gather_body(idx_hbm, table_hbm, out_hbm, idx_v, rows_v, sem):
    wid = lax.axis_index("s") * 2 + lax.axis_index("c")
    base = wid * ROWS_PER_W
    pltpu.sync_copy(idx_hbm.at[pl.ds(base, ROWS_PER_W)], idx_v)

    def step(i, carry):
        s = i * UNROLL
        vec = idx_v[pl.ds(s, UNROLL)]
        cps = []
        for j in range(UNROLL):
            r = vec[j]
            cps.append(pltpu.async_copy(
                table_hbm.at[pl.ds(r, 1)], rows_v.at[pl.ds(s + j, 1)], sem))
        for cp in cps:
            cp.wait()
        return carry

    lax.fori_loop(0, ROWS_PER_W // UNROLL, step, 0)
    pltpu.sync_copy(rows_v, out_hbm.at[pl.ds(base, ROWS_PER_W)])


def _make_sc_gather():
    return pl.kernel(
        _sc_gather_body,
        out_type=jax.ShapeDtypeStruct((B, EMB), jnp.float32),
        mesh=plsc.VectorSubcoreMesh(core_axis_name="c", subcore_axis_name="s"),
        scratch_types=[
            pltpu.VMEM((ROWS_PER_W,), jnp.int32),
            pltpu.VMEM((ROWS_PER_W, EMB), jnp.float32),
            pltpu.SemaphoreType.DMA,
        ],
    )


def _tc_body(id_emb_ref, sec_ref, stg_ref, reg_ref, numT_ref, tbdT_ref,
             w1aT_ref, w1mT_ref, w1nT_ref, b1_ref, g1_ref, be1_ref,
             w2T_ref, b2_ref, g2_ref, be2_ref, outT_ref):
    f32 = jnp.float32
    id_emb = id_emb_ref[:]
    iota = lax.broadcasted_iota(jnp.int32, (OH, B), 0)
    ohT = (jnp.where(iota == sec_ref[:], 1.0, 0.0)
           + jnp.where(iota == stg_ref[:], 1.0, 0.0)
           + jnp.where(iota == reg_ref[:], 1.0, 0.0)).astype(f32)
    mT = jnp.dot(w1mT_ref[:], tbdT_ref[:], preferred_element_type=f32)
    p1 = (lax.dot_general(w1aT_ref[:], id_emb, (((1,), (1,)), ((), ())),
                          preferred_element_type=f32)
          + jnp.dot(mT, ohT, preferred_element_type=f32)
          + jnp.dot(w1nT_ref[:], numT_ref[:], preferred_element_type=f32)
          + b1_ref[:])
    h = jnp.maximum(p1, 0.0)
    mu = jnp.mean(h, axis=1, keepdims=True)
    var = jnp.mean((h - mu) * (h - mu), axis=1, keepdims=True)
    h = (h - mu) / jnp.sqrt(var + 1e-5) * g1_ref[:] + be1_ref[:]
    p2 = jnp.dot(w2T_ref[:], h, preferred_element_type=f32) + b2_ref[:]
    h2 = jnp.maximum(p2, 0.0)
    mu2 = jnp.mean(h2, axis=1, keepdims=True)
    var2 = jnp.mean((h2 - mu2) * (h2 - mu2), axis=1, keepdims=True)
    h2 = (h2 - mu2) / jnp.sqrt(var2 + 1e-5) * g2_ref[:] + be2_ref[:]
    nrm = jnp.sqrt(jnp.sum(h2 * h2, axis=0, keepdims=True))
    outT_ref[:] = h2 / jnp.maximum(nrm, 1e-12)


_tc_mlp = pl.pallas_call(
    _tc_body,
    out_shape=jax.ShapeDtypeStruct((EMB, B), jnp.float32),
)


def kernel(id, sector, stage, region, deal_size, revenue_multiple, growth_rate,
           profitability, team_experience, market_size, deal_table,
           sector_table, stage_table, region_table, W1, b1, g1, be1,
           W2, b2, g2, be2):
    id_emb = _make_sc_gather()(id.astype(jnp.int32), deal_table)

    numT = jnp.stack([deal_size, revenue_multiple, growth_rate, profitability,
                      team_experience, market_size], axis=0).astype(jnp.float32)
    numT = jnp.pad(numT, ((0, 2), (0, 0)))
    w1nT = jnp.pad(W1[112:118], ((0, 2), (0, 0))).T

    tbdT = jnp.zeros((48, OH), dtype=jnp.float32)
    tbdT = tbdT.at[0:16, 0:50].set(sector_table.T)
    tbdT = tbdT.at[16:32, 50:60].set(stage_table.T)
    tbdT = tbdT.at[32:48, 60:80].set(region_table.T)

    sec = sector.astype(jnp.int32).reshape(1, B)
    stg = stage.astype(jnp.int32).reshape(1, B) + 50
    reg = region.astype(jnp.int32).reshape(1, B) + 60

    outT = _tc_mlp(
        id_emb, sec, stg, reg, numT, tbdT,
        W1[0:64].T, W1[64:112].T, w1nT,
        b1.reshape(128, 1), g1.reshape(128, 1), be1.reshape(128, 1),
        W2.T, b2.reshape(64, 1), g2.reshape(64, 1), be2.reshape(64, 1),
    )
    return outT.T
